# Initial kernel scaffold; baseline (speedup 1.0000x reference)
import jax
import jax.numpy as jnp
from jax.experimental import pallas as pl


def kernel(input_user, input_movie, Eu, Em, Eub, Emb, W1, b1, alpha, gamma, beta, mmean, mvar, W2, b2):
    raise NotImplementedError("write your pallas kernel here")


# speedup vs baseline: 2.1966x; 2.1966x over previous
import functools

import jax
import jax.numpy as jnp
from jax import lax
from jax.experimental import pallas as pl
from jax.experimental.pallas import tpu as pltpu
from jax.experimental.pallas import tpu_sc as plsc

NC = 2
NS = 16
NW = NC * NS
CH = 128
BN_EPS = 1e-3


def _make_gather(B, K):
    b_per_w = B // NW
    n_chunks = b_per_w // CH
    mesh = plsc.VectorSubcoreMesh(core_axis_name="c", subcore_axis_name="s")

    @functools.partial(
        pl.kernel,
        mesh=mesh,
        out_type=jax.ShapeDtypeStruct((4, B, K), jnp.float32),
        scratch_types=[
            pltpu.VMEM((CH,), jnp.int32),
            pltpu.VMEM((CH, K), jnp.float32),
            pltpu.SemaphoreType.DMA,
        ],
    )
    def gather_kernel(u_hbm, m_hbm, Eu, Em, Eub, Emb, out, idx_v, rows_v, sem):
        wid = lax.axis_index("s") * NC + lax.axis_index("c")
        base = wid * b_per_w
        for t, (table, idx_hbm) in enumerate(
            ((Eu, u_hbm), (Em, m_hbm), (Eub, u_hbm), (Emb, m_hbm))
        ):
            for c in range(n_chunks):
                off = base + c * CH
                pltpu.sync_copy(idx_hbm.at[pl.ds(off, CH)], idx_v)
                pltpu.async_copy(table.at[idx_v], rows_v, sem).wait()
                pltpu.sync_copy(rows_v, out.at[t, pl.ds(off, CH)])

    return gather_kernel


def _dense_body(g_ref, w1_ref, b1_ref, sp_ref, sn_ref, t_ref, w2_ref,
                c_ref, o_ref):
    def branch(i):
        y = jnp.dot(g_ref[i], w1_ref[i], preferred_element_type=jnp.float32,
                    precision=lax.Precision.HIGHEST)
        y = y + b1_ref[i]
        return jnp.where(y >= 0, y * sp_ref[i], y * sn_ref[i]) + t_ref[i]

    z0 = branch(0)
    z1 = branch(1)
    z2 = branch(2)
    z3 = branch(3)
    dot = jnp.sum(z0 * z1, axis=1, keepdims=True)
    bias = jnp.sum(z2 * w2_ref[0], axis=1, keepdims=True)
    bias += jnp.sum(z3 * w2_ref[1], axis=1, keepdims=True)
    o_ref[...] = dot + bias + c_ref[0, 0]


def kernel(input_user, input_movie, Eu, Em, Eub, Emb, W1, b1, alpha, gamma,
           beta, mmean, mvar, W2, b2):
    B = input_user.shape[0]
    K = Eu.shape[1]
    H = W1.shape[2]

    u = input_user[:, 0].astype(jnp.int32)
    m = input_movie[:, 0].astype(jnp.int32)

    gathered = _make_gather(B, K)(u, m, Eu, Em, Eub, Emb)

    s = gamma * lax.rsqrt(mvar + BN_EPS)
    sn = alpha * s
    t = beta - mmean * s
    w2f = jnp.concatenate([W2[0], W2[1]], axis=1).T
    cbias = (b2[0, 0] + b2[1, 0]).reshape(1, 1)

    R = 512
    out = pl.pallas_call(
        _dense_body,
        grid=(B // R,),
        in_specs=[
            pl.BlockSpec((4, R, K), lambda i: (0, i, 0)),
            pl.BlockSpec((4, K, H), lambda i: (0, 0, 0)),
            pl.BlockSpec((4, H), lambda i: (0, 0)),
            pl.BlockSpec((4, H), lambda i: (0, 0)),
            pl.BlockSpec((4, H), lambda i: (0, 0)),
            pl.BlockSpec((4, H), lambda i: (0, 0)),
            pl.BlockSpec((2, H), lambda i: (0, 0)),
            pl.BlockSpec((1, 1), lambda i: (0, 0), memory_space=pltpu.SMEM),
        ],
        out_specs=pl.BlockSpec((R, 1), lambda i: (i, 0)),
        out_shape=jax.ShapeDtypeStruct((B, 1), jnp.float32),
    )(gathered, W1, b1, s, sn, t, w2f, cbias)
    return out


# speedup vs baseline: 2.7208x; 1.2386x over previous
import functools

import jax
import jax.numpy as jnp
from jax import lax
from jax.experimental import pallas as pl
from jax.experimental.pallas import tpu as pltpu
from jax.experimental.pallas import tpu_sc as plsc

NC = 2
NS = 16
NW = NC * NS
CH = 128
BN_EPS = 1e-3


def _make_gather(B, K):
    b_per_w = B // NW
    n_chunks = b_per_w // CH
    n_tasks = 4 * n_chunks
    mesh = plsc.VectorSubcoreMesh(core_axis_name="c", subcore_axis_name="s")

    @functools.partial(
        pl.kernel,
        mesh=mesh,
        out_type=jax.ShapeDtypeStruct((4, B, K), jnp.float32),
        scratch_types=[
            pltpu.VMEM((b_per_w,), jnp.int32),
            pltpu.VMEM((b_per_w,), jnp.int32),
            pltpu.VMEM((CH, K), jnp.float32),
            pltpu.VMEM((CH, K), jnp.float32),
            pltpu.SemaphoreType.DMA,
            pltpu.SemaphoreType.DMA,
            pltpu.SemaphoreType.DMA,
            pltpu.SemaphoreType.DMA,
        ],
    )
    def gather_kernel(u_hbm, m_hbm, Eu, Em, Eub, Emb, out,
                      idxu, idxm, rows0, rows1, g0, g1, s0, s1):
        wid = lax.axis_index("s") * NC + lax.axis_index("c")
        base = wid * b_per_w
        pltpu.sync_copy(u_hbm.at[pl.ds(base, b_per_w)], idxu)
        pltpu.sync_copy(m_hbm.at[pl.ds(base, b_per_w)], idxm)
        tables = (Eu, Em, Eub, Emb)
        idxs = (idxu, idxm, idxu, idxm)
        rows = (rows0, rows1)
        gsem = (g0, g1)
        ssem = (s0, s1)

        def task(t):
            ti, ci = divmod(t, n_chunks)
            return tables[ti], idxs[ti], ti, ci

        def fire_gather(t):
            table, idx, ti, ci = task(t)
            b = t % 2
            return pltpu.async_copy(
                table.at[idx.at[pl.ds(ci * CH, CH)]], rows[b], gsem[b])

        def fire_scatter(t):
            _, _, ti, ci = task(t)
            b = t % 2
            return pltpu.async_copy(
                rows[b], out.at[ti, pl.ds(base + ci * CH, CH)], ssem[b])

        gathers = [None] * n_tasks
        scatters = [None] * n_tasks
        gathers[0] = fire_gather(0)
        for t in range(1, n_tasks):
            if t >= 2:
                scatters[t - 2].wait()
            gathers[t] = fire_gather(t)
            gathers[t - 1].wait()
            scatters[t - 1] = fire_scatter(t - 1)
        gathers[n_tasks - 1].wait()
        scatters[n_tasks - 1] = fire_scatter(n_tasks - 1)
        scatters[n_tasks - 2].wait()
        scatters[n_tasks - 1].wait()

    return gather_kernel


def _dense_body(g_ref, w1_ref, b1_ref, sp_ref, sn_ref, t_ref, w2_ref,
                c_ref, o_ref):
    def branch(i):
        y = jnp.dot(g_ref[i], w1_ref[i], preferred_element_type=jnp.float32,
                    precision=lax.Precision.HIGHEST)
        y = y + b1_ref[i]
        return jnp.where(y >= 0, y * sp_ref[i], y * sn_ref[i]) + t_ref[i]

    z0 = branch(0)
    z1 = branch(1)
    z2 = branch(2)
    z3 = branch(3)
    dot = jnp.sum(z0 * z1, axis=1, keepdims=True)
    bias = jnp.sum(z2 * w2_ref[0], axis=1, keepdims=True)
    bias += jnp.sum(z3 * w2_ref[1], axis=1, keepdims=True)
    o_ref[...] = dot + bias + c_ref[0, 0]


def kernel(input_user, input_movie, Eu, Em, Eub, Emb, W1, b1, alpha, gamma,
           beta, mmean, mvar, W2, b2):
    B = input_user.shape[0]
    K = Eu.shape[1]
    H = W1.shape[2]

    u = input_user[:, 0].astype(jnp.int32)
    m = input_movie[:, 0].astype(jnp.int32)

    gathered = _make_gather(B, K)(u, m, Eu, Em, Eub, Emb)

    s = gamma * lax.rsqrt(mvar + BN_EPS)
    sn = alpha * s
    t = beta - mmean * s
    w2f = jnp.concatenate([W2[0], W2[1]], axis=1).T
    cbias = (b2[0, 0] + b2[1, 0]).reshape(1, 1)

    R = 1024
    out = pl.pallas_call(
        _dense_body,
        grid=(B // R,),
        in_specs=[
            pl.BlockSpec((4, R, K), lambda i: (0, i, 0)),
            pl.BlockSpec((4, K, H), lambda i: (0, 0, 0)),
            pl.BlockSpec((4, H), lambda i: (0, 0)),
            pl.BlockSpec((4, H), lambda i: (0, 0)),
            pl.BlockSpec((4, H), lambda i: (0, 0)),
            pl.BlockSpec((4, H), lambda i: (0, 0)),
            pl.BlockSpec((2, H), lambda i: (0, 0)),
            pl.BlockSpec((1, 1), lambda i: (0, 0), memory_space=pltpu.SMEM),
        ],
        out_specs=pl.BlockSpec((R, 1), lambda i: (i, 0)),
        out_shape=jax.ShapeDtypeStruct((B, 1), jnp.float32),
    )(gathered, W1, b1, s, sn, t, w2f, cbias)
    return out


# speedup vs baseline: 3.1453x; 1.1560x over previous
import functools

import jax
import jax.numpy as jnp
from jax import lax
from jax.experimental import pallas as pl
from jax.experimental.pallas import tpu as pltpu
from jax.experimental.pallas import tpu_sc as plsc

NC = 2
NS = 16
NW = NC * NS
CH = 128
BN_EPS = 1e-3


def _make_gather(B, K):
    b_per_w = B // NW
    n_chunks = b_per_w // CH
    n_tasks = 4 * n_chunks
    mesh = plsc.VectorSubcoreMesh(core_axis_name="c", subcore_axis_name="s")

    @functools.partial(
        pl.kernel,
        mesh=mesh,
        out_type=jax.ShapeDtypeStruct((4, B, K), jnp.float32),
        scratch_types=[
            pltpu.VMEM((b_per_w,), jnp.int32),
            pltpu.VMEM((b_per_w,), jnp.int32),
            pltpu.VMEM((CH, K), jnp.float32),
            pltpu.VMEM((CH, K), jnp.float32),
            pltpu.SemaphoreType.DMA,
            pltpu.SemaphoreType.DMA,
            pltpu.SemaphoreType.DMA,
            pltpu.SemaphoreType.DMA,
        ],
    )
    def gather_kernel(u_hbm, m_hbm, Eu, Em, Eub, Emb, out,
                      idxu, idxm, rows0, rows1, g0, g1, s0, s1):
        wid = lax.axis_index("s") * NC + lax.axis_index("c")
        base = wid * b_per_w
        pltpu.sync_copy(u_hbm.at[pl.ds(base, b_per_w)], idxu)
        pltpu.sync_copy(m_hbm.at[pl.ds(base, b_per_w)], idxm)
        tables = (Eu, Em, Eub, Emb)
        idxs = (idxu, idxm, idxu, idxm)
        rows = (rows0, rows1)
        gsem = (g0, g1)
        ssem = (s0, s1)

        def task(t):
            ti, ci = divmod(t, n_chunks)
            return tables[ti], idxs[ti], ti, ci

        def fire_gather(t):
            table, idx, ti, ci = task(t)
            b = t % 2
            return pltpu.async_copy(
                table.at[idx.at[pl.ds(ci * CH, CH)]], rows[b], gsem[b])

        def fire_scatter(t):
            _, _, ti, ci = task(t)
            b = t % 2
            return pltpu.async_copy(
                rows[b], out.at[ti, pl.ds(base + ci * CH, CH)], ssem[b])

        gathers = [None] * n_tasks
        scatters = [None] * n_tasks
        gathers[0] = fire_gather(0)
        for t in range(1, n_tasks):
            if t >= 2:
                scatters[t - 2].wait()
            gathers[t] = fire_gather(t)
            gathers[t - 1].wait()
            scatters[t - 1] = fire_scatter(t - 1)
        gathers[n_tasks - 1].wait()
        scatters[n_tasks - 1] = fire_scatter(n_tasks - 1)
        scatters[n_tasks - 2].wait()
        scatters[n_tasks - 1].wait()

    return gather_kernel


def _dense_body(g_ref, w1h_ref, w1l_ref, b1_ref, sp_ref, sn_ref, t_ref,
                w2_ref, c_ref, o_ref):
    def branch(i):
        x = g_ref[i]
        xh = x.astype(jnp.bfloat16)
        xl = (x - xh.astype(jnp.float32)).astype(jnp.bfloat16)
        y = jnp.dot(xh, w1h_ref[i], preferred_element_type=jnp.float32)
        y += jnp.dot(xh, w1l_ref[i], preferred_element_type=jnp.float32)
        y += jnp.dot(xl, w1h_ref[i], preferred_element_type=jnp.float32)
        y = y + b1_ref[i]
        return jnp.where(y >= 0, y * sp_ref[i], y * sn_ref[i]) + t_ref[i]

    z0 = branch(0)
    z1 = branch(1)
    z2 = branch(2)
    z3 = branch(3)
    dot = jnp.sum(z0 * z1, axis=1, keepdims=True)
    bias = jnp.sum(z2 * w2_ref[0], axis=1, keepdims=True)
    bias += jnp.sum(z3 * w2_ref[1], axis=1, keepdims=True)
    o_ref[...] = dot + bias + c_ref[0, 0]


def kernel(input_user, input_movie, Eu, Em, Eub, Emb, W1, b1, alpha, gamma,
           beta, mmean, mvar, W2, b2):
    B = input_user.shape[0]
    K = Eu.shape[1]
    H = W1.shape[2]

    u = input_user[:, 0].astype(jnp.int32)
    m = input_movie[:, 0].astype(jnp.int32)

    gathered = _make_gather(B, K)(u, m, Eu, Em, Eub, Emb)

    s = gamma * lax.rsqrt(mvar + BN_EPS)
    sn = alpha * s
    t = beta - mmean * s
    w2f = jnp.concatenate([W2[0], W2[1]], axis=1).T
    cbias = (b2[0, 0] + b2[1, 0]).reshape(1, 1)
    W1h = W1.astype(jnp.bfloat16)
    W1l = (W1 - W1h.astype(jnp.float32)).astype(jnp.bfloat16)

    R = 1024
    out = pl.pallas_call(
        _dense_body,
        grid=(B // R,),
        in_specs=[
            pl.BlockSpec((4, R, K), lambda i: (0, i, 0)),
            pl.BlockSpec((4, K, H), lambda i: (0, 0, 0)),
            pl.BlockSpec((4, K, H), lambda i: (0, 0, 0)),
            pl.BlockSpec((4, H), lambda i: (0, 0)),
            pl.BlockSpec((4, H), lambda i: (0, 0)),
            pl.BlockSpec((4, H), lambda i: (0, 0)),
            pl.BlockSpec((4, H), lambda i: (0, 0)),
            pl.BlockSpec((2, H), lambda i: (0, 0)),
            pl.BlockSpec((1, 1), lambda i: (0, 0), memory_space=pltpu.SMEM),
        ],
        out_specs=pl.BlockSpec((R, 1), lambda i: (i, 0)),
        out_shape=jax.ShapeDtypeStruct((B, 1), jnp.float32),
    )(gathered, W1h, W1l, b1, s, sn, t, w2f, cbias)
    return out


# speedup vs baseline: 3.2879x; 1.0453x over previous
import functools

import jax
import jax.numpy as jnp
from jax import lax
from jax.experimental import pallas as pl
from jax.experimental.pallas import tpu as pltpu
from jax.experimental.pallas import tpu_sc as plsc

NC = 2
NS = 16
NW = NC * NS
CH = 128
BN_EPS = 1e-3


def _make_gather(B, K):
    b_per_w = B // NW
    n_chunks = b_per_w // CH
    n_tasks = 4 * n_chunks
    mesh = plsc.VectorSubcoreMesh(core_axis_name="c", subcore_axis_name="s")

    @functools.partial(
        pl.kernel,
        mesh=mesh,
        out_type=jax.ShapeDtypeStruct((4, B, K), jnp.float32),
        scratch_types=[
            pltpu.VMEM((b_per_w,), jnp.int32),
            pltpu.VMEM((b_per_w,), jnp.int32),
            pltpu.VMEM((CH, K), jnp.float32),
            pltpu.VMEM((CH, K), jnp.float32),
            pltpu.SemaphoreType.DMA,
            pltpu.SemaphoreType.DMA,
            pltpu.SemaphoreType.DMA,
            pltpu.SemaphoreType.DMA,
        ],
    )
    def gather_kernel(u_hbm, m_hbm, Eu, Em, Eub, Emb, out,
                      idxu, idxm, rows0, rows1, g0, g1, s0, s1):
        wid = lax.axis_index("s") * NC + lax.axis_index("c")
        base = wid * b_per_w
        pltpu.sync_copy(u_hbm.at[pl.ds(base, b_per_w)], idxu)
        pltpu.sync_copy(m_hbm.at[pl.ds(base, b_per_w)], idxm)
        tables = (Eu, Em, Eub, Emb)
        idxs = (idxu, idxm, idxu, idxm)
        rows = (rows0, rows1)
        gsem = (g0, g1)
        ssem = (s0, s1)

        def task(t):
            ti, ci = divmod(t, n_chunks)
            return tables[ti], idxs[ti], ti, ci

        def fire_gather(t):
            table, idx, ti, ci = task(t)
            b = t % 2
            return pltpu.async_copy(
                table.at[idx.at[pl.ds(ci * CH, CH)]], rows[b], gsem[b])

        def fire_scatter(t):
            _, _, ti, ci = task(t)
            b = t % 2
            return pltpu.async_copy(
                rows[b], out.at[ti, pl.ds(base + ci * CH, CH)], ssem[b])

        gathers = [None] * n_tasks
        scatters = [None] * n_tasks
        gathers[0] = fire_gather(0)
        for t in range(1, n_tasks):
            if t >= 2:
                scatters[t - 2].wait()
            gathers[t] = fire_gather(t)
            gathers[t - 1].wait()
            scatters[t - 1] = fire_scatter(t - 1)
        gathers[n_tasks - 1].wait()
        scatters[n_tasks - 1] = fire_scatter(n_tasks - 1)
        scatters[n_tasks - 2].wait()
        scatters[n_tasks - 1].wait()

    return gather_kernel


def _dense_body(g_ref, w1h_ref, w1l_ref, b1_ref, sp_ref, sn_ref, t_ref,
                w2_ref, c_ref, o_ref):
    def branch(i):
        x = g_ref[i]
        xh = x.astype(jnp.bfloat16)
        xl = (x - xh.astype(jnp.float32)).astype(jnp.bfloat16)
        y = jnp.dot(xh, w1h_ref[i], preferred_element_type=jnp.float32)
        y += jnp.dot(xh, w1l_ref[i], preferred_element_type=jnp.float32)
        y += jnp.dot(xl, w1h_ref[i], preferred_element_type=jnp.float32)
        y = y + b1_ref[i]
        return jnp.where(y >= 0, y * sp_ref[i], y * sn_ref[i]) + t_ref[i]

    z0 = branch(0)
    z1 = branch(1)
    z2 = branch(2)
    z3 = branch(3)
    dot = jnp.sum(z0 * z1, axis=1, keepdims=True)
    bias = jnp.sum(z2 * w2_ref[0], axis=1, keepdims=True)
    bias += jnp.sum(z3 * w2_ref[1], axis=1, keepdims=True)
    o_ref[...] = dot + bias + c_ref[0, 0]


def kernel(input_user, input_movie, Eu, Em, Eub, Emb, W1, b1, alpha, gamma,
           beta, mmean, mvar, W2, b2):
    B = input_user.shape[0]
    K = Eu.shape[1]
    H = W1.shape[2]

    u = input_user[:, 0].astype(jnp.int32)
    m = input_movie[:, 0].astype(jnp.int32)

    s = gamma * lax.rsqrt(mvar + BN_EPS)
    sn = alpha * s
    t = beta - mmean * s
    w2f = jnp.concatenate([W2[0], W2[1]], axis=1).T
    cbias = (b2[0, 0] + b2[1, 0]).reshape(1, 1)
    W1h = W1.astype(jnp.bfloat16)
    W1l = (W1 - W1h.astype(jnp.float32)).astype(jnp.bfloat16)

    NCHUNK = 2
    Bc = B // NCHUNK
    R = 1024
    gather_fn = _make_gather(Bc, K)

    def dense_fn(g):
        return pl.pallas_call(
            _dense_body,
            grid=(Bc // R,),
            in_specs=[
                pl.BlockSpec((4, R, K), lambda i: (0, i, 0)),
                pl.BlockSpec((4, K, H), lambda i: (0, 0, 0)),
                pl.BlockSpec((4, K, H), lambda i: (0, 0, 0)),
                pl.BlockSpec((4, H), lambda i: (0, 0)),
                pl.BlockSpec((4, H), lambda i: (0, 0)),
                pl.BlockSpec((4, H), lambda i: (0, 0)),
                pl.BlockSpec((4, H), lambda i: (0, 0)),
                pl.BlockSpec((2, H), lambda i: (0, 0)),
                pl.BlockSpec((1, 1), lambda i: (0, 0),
                             memory_space=pltpu.SMEM),
            ],
            out_specs=pl.BlockSpec((R, 1), lambda i: (i, 0)),
            out_shape=jax.ShapeDtypeStruct((Bc, 1), jnp.float32),
        )(g, W1h, W1l, b1, s, sn, t, w2f, cbias)

    outs = []
    for c in range(NCHUNK):
        sl = slice(c * Bc, (c + 1) * Bc)
        g = gather_fn(u[sl], m[sl], Eu, Em, Eub, Emb)
        outs.append(dense_fn(g))
    return jnp.concatenate(outs, axis=0)
